# baseline (speedup 1.0000x reference)
import functools

import jax
import jax.numpy as jnp
from jax import lax
from jax.experimental import pallas as pl
from jax.experimental.pallas import tpu as pltpu
from jax.experimental.pallas import tpu_sc as plsc

_H, _B, _N, _D, _C = 1, 64, 576, 256, 1024
_ROWS = _B * _N
_TILE = 256
_GRID = _ROWS // _TILE

_NC, _NS = 2, 16
_NW = _NC * _NS
_BPW = _ROWS // _NW
_CHUNK = 192
_NCHUNK = _BPW // _CHUNK


def _dist_argmax_kernel(x_ref, e_ref, dist_ref, ind_ref):
    x = x_ref[...]
    e = e_ref[...]
    dist = jax.lax.dot_general(
        x, e, (((1,), (1,)), ((), ())), preferred_element_type=jnp.float32)
    dist_ref[...] = dist
    ind_ref[0, 0, :] = jnp.argmax(dist, axis=1).astype(jnp.int32)


_SC_MESH = plsc.VectorSubcoreMesh(core_axis_name="c", subcore_axis_name="s")


@functools.partial(
    pl.kernel,
    mesh=_SC_MESH,
    out_type=jax.ShapeDtypeStruct((_ROWS, _D), jnp.float32),
    scratch_types=[
        pltpu.VMEM((_CHUNK,), jnp.int32),
        pltpu.VMEM((_CHUNK, _D), jnp.float32),
        pltpu.SemaphoreType.DMA,
    ],
)
def _sc_gather(ind_hbm, table_hbm, out_hbm, idx_v, rows_v, sem):
    wid = lax.axis_index("s") * _NC + lax.axis_index("c")
    base = wid * _BPW

    def body(i, carry):
        off = base + i * _CHUNK
        pltpu.sync_copy(ind_hbm.at[pl.ds(off, _CHUNK)], idx_v)
        pltpu.async_copy(table_hbm.at[idx_v], rows_v, sem).wait()
        pltpu.sync_copy(rows_v, out_hbm.at[pl.ds(off, _CHUNK)])
        return carry

    lax.fori_loop(0, _NCHUNK, body, 0)


def kernel(x, embed):
    x = x.astype(jnp.float32)
    xf = x.reshape(_ROWS, _D)
    e = embed.reshape(_C, _D)
    dist, ind3 = pl.pallas_call(
        _dist_argmax_kernel,
        grid=(_GRID,),
        in_specs=[
            pl.BlockSpec((_TILE, _D), lambda i: (i, 0)),
            pl.BlockSpec((_C, _D), lambda i: (0, 0)),
        ],
        out_specs=[
            pl.BlockSpec((_TILE, _C), lambda i: (i, 0)),
            pl.BlockSpec((1, 1, _TILE), lambda i: (i, 0, 0)),
        ],
        out_shape=[
            jax.ShapeDtypeStruct((_ROWS, _C), jnp.float32),
            jax.ShapeDtypeStruct((_GRID, 1, _TILE), jnp.int32),
        ],
    )(xf, e)
    ind_flat = ind3.reshape(_ROWS)
    quant = _sc_gather(ind_flat, e)
    quantize = quant.reshape(_B, _N, _D)
    embed_ind = ind3.reshape(_B, _N)
    dist_out = dist.reshape(_H, _B, _N, _C)
    return (quantize, embed_ind, dist_out)


# speedup vs baseline: 1.0230x; 1.0230x over previous
import functools

import jax
import jax.numpy as jnp
from jax import lax
from jax.experimental import pallas as pl
from jax.experimental.pallas import tpu as pltpu
from jax.experimental.pallas import tpu_sc as plsc

_H, _B, _N, _D, _C = 1, 64, 576, 256, 1024
_ROWS = _B * _N
_TILE = 256
_GRID = _ROWS // _TILE

_NC, _NS = 2, 16
_NW = _NC * _NS
_BPW = _ROWS // _NW
_CHUNK = 192
_NCHUNK = _BPW // _CHUNK


def _dist_argmax_kernel(x_ref, e_ref, dist_ref, ind_ref):
    x = x_ref[...]
    e = e_ref[...]
    dist = jax.lax.dot_general(
        x, e, (((1,), (1,)), ((), ())), preferred_element_type=jnp.float32)
    dist_ref[...] = dist
    ind_ref[0, 0, :] = jnp.argmax(dist, axis=1).astype(jnp.int32)


_SC_MESH = plsc.VectorSubcoreMesh(core_axis_name="c", subcore_axis_name="s")


@functools.partial(
    pl.kernel,
    mesh=_SC_MESH,
    out_type=jax.ShapeDtypeStruct((_ROWS, _D), jnp.float32),
    scratch_types=[
        pltpu.VMEM((_BPW,), jnp.int32),
        pltpu.VMEM((_CHUNK, _D), jnp.float32),
        pltpu.VMEM((_CHUNK, _D), jnp.float32),
        pltpu.SemaphoreType.DMA,
        pltpu.SemaphoreType.DMA,
    ],
)
def _sc_gather(ind_hbm, table_hbm, out_hbm, idx_v, rows0, rows1, sem0, sem1):
    wid = lax.axis_index("s") * _NC + lax.axis_index("c")
    base = wid * _BPW
    pltpu.sync_copy(ind_hbm.at[pl.ds(base, _BPW)], idx_v)
    bufs = (rows0, rows1)
    sems = (sem0, sem1)
    copies = []
    for i in range(_NCHUNK):
        cp = pltpu.async_copy(
            table_hbm.at[idx_v.at[pl.ds(i * _CHUNK, _CHUNK)]],
            bufs[i % 2], sems[i % 2])
        copies.append(cp)
        if i > 0:
            copies[i - 1].wait()
            pltpu.sync_copy(bufs[(i - 1) % 2],
                            out_hbm.at[pl.ds(base + (i - 1) * _CHUNK, _CHUNK)])
    copies[_NCHUNK - 1].wait()
    pltpu.sync_copy(bufs[(_NCHUNK - 1) % 2],
                    out_hbm.at[pl.ds(base + (_NCHUNK - 1) * _CHUNK, _CHUNK)])


def kernel(x, embed):
    x = x.astype(jnp.float32)
    xf = x.reshape(_ROWS, _D)
    e = embed.reshape(_C, _D)
    dist, ind3 = pl.pallas_call(
        _dist_argmax_kernel,
        grid=(_GRID,),
        in_specs=[
            pl.BlockSpec((_TILE, _D), lambda i: (i, 0)),
            pl.BlockSpec((_C, _D), lambda i: (0, 0)),
        ],
        out_specs=[
            pl.BlockSpec((_TILE, _C), lambda i: (i, 0)),
            pl.BlockSpec((1, 1, _TILE), lambda i: (i, 0, 0)),
        ],
        out_shape=[
            jax.ShapeDtypeStruct((_ROWS, _C), jnp.float32),
            jax.ShapeDtypeStruct((_GRID, 1, _TILE), jnp.int32),
        ],
    )(xf, e)
    ind_flat = ind3.reshape(_ROWS)
    quant = _sc_gather(ind_flat, e)
    quantize = quant.reshape(_B, _N, _D)
    embed_ind = ind3.reshape(_B, _N)
    dist_out = dist.reshape(_H, _B, _N, _C)
    return (quantize, embed_ind, dist_out)
